# Initial kernel scaffold; baseline (speedup 1.0000x reference)
import jax
import jax.numpy as jnp
from jax.experimental import pallas as pl


def kernel(lbls, no_lbl_idx, knn_sc, knn_fc, train_idx):
    raise NotImplementedError("write your pallas kernel here")


# speedup vs baseline: 239.3328x; 239.3328x over previous
import functools

import jax
import jax.numpy as jnp
from jax import lax
from jax.experimental import pallas as pl
from jax.experimental.pallas import tpu as pltpu
from jax.experimental.pallas import tpu_sc as plsc

N_PAD = 102400
ROWS = N_PAD // 128
GROUP = 128
R = 4
NC, NS = 2, 16
NW = NC * NS


def _prep_body(lbls_ref, train_ref, null_ref, val_ref, af_ref):
    a = train_ref[...] * (1.0 - null_ref[...])
    val_ref[...] = lbls_ref[...] * a
    af_ref[...] = a


def _fin_body(lbls_ref, train_ref, null_ref,
              s1a_ref, s1b_ref, c1a_ref, c1b_ref,
              s2a_ref, s2b_ref, c2a_ref, c2b_ref,
              out_ref, mask_ref):
    lbls = lbls_ref[...]
    train = train_ref[...] > 0.0
    b = train & (null_ref[...] > 0.0)

    def mean_or_lbls(s, c):
        has = b & (c > 0.0)
        return jnp.where(has, s / jnp.maximum(c, 1.0), lbls)

    l1 = mean_or_lbls(s1a_ref[...] + s1b_ref[...], c1a_ref[...] + c1b_ref[...])
    l2 = mean_or_lbls(s2a_ref[...] + s2b_ref[...], c2a_ref[...] + c2b_ref[...])
    fin = (jnp.abs(l1) != jnp.inf) & (jnp.abs(l2) != jnp.inf) & train
    out_ref[...] = jnp.where(fin, (l1 + l2) * 0.5, lbls)
    mask_ref[...] = fin.astype(jnp.int32)


def _sc_body(val_t, af_t, src1, dst1, src2, dst2, zeros,
             out1, out2,
             acc_s1, acc_c1, acc_s2, acc_c2,
             src_buf, dst_buf, vrows, arows, zbuf, sem_g, sem_s):
    cid = lax.axis_index("c")
    sid = lax.axis_index("s")
    wid = sid * NC + cid

    per_tile_nodes = N_PAD // NS
    zslice = pl.ds(sid * per_tile_nodes, per_tile_nodes)

    pltpu.sync_copy(zeros, zbuf)
    for acc in (acc_s1, acc_c1, acc_s2, acc_c2):
        pltpu.sync_copy(zbuf, acc.at[zslice])
    plsc.subcore_barrier()

    def run_graph(srcg, dstg, acc_s, acc_c):
        n_chunks_total = srcg.shape[0] // R
        per_tile = n_chunks_total // NW
        rem = n_chunks_total - per_tile * NW
        base = wid * per_tile + jnp.minimum(wid, rem)
        n = per_tile + (wid < rem).astype(jnp.int32)

        def chunk_body(i, carry):
            off = (base + i) * R
            pltpu.sync_copy(srcg.at[pl.ds(off, R)], src_buf)
            pltpu.sync_copy(dstg.at[pl.ds(off, R)], dst_buf)
            g = []
            for j in range(R):
                g.append(pltpu.async_copy(val_t.at[src_buf.at[j]],
                                          vrows.at[j], sem_g))
                g.append(pltpu.async_copy(af_t.at[src_buf.at[j]],
                                          arows.at[j], sem_g))
            for d in g:
                d.wait()
            s = []
            for j in range(R):
                s.append(pltpu.async_copy(vrows.at[j], acc_s.at[dst_buf.at[j]],
                                          sem_s, add=True))
                s.append(pltpu.async_copy(arows.at[j], acc_c.at[dst_buf.at[j]],
                                          sem_s, add=True))
            for d in s:
                d.wait()
            return carry

        lax.fori_loop(0, n, chunk_body, 0)

    run_graph(src1, dst1, acc_s1, acc_c1)
    run_graph(src2, dst2, acc_s2, acc_c2)
    plsc.subcore_barrier()

    for k, acc in enumerate((acc_s1, acc_c1)):
        pltpu.sync_copy(acc.at[zslice], zbuf)
        pltpu.sync_copy(zbuf, out1.at[cid, k, zslice])
    for k, acc in enumerate((acc_s2, acc_c2)):
        pltpu.sync_copy(acc.at[zslice], zbuf)
        pltpu.sync_copy(zbuf, out2.at[cid, k, zslice])


_SC_KERNEL_CACHE = []


def _get_sc_kernel():
    if _SC_KERNEL_CACHE:
        return _SC_KERNEL_CACHE[0]
    k = functools.partial(
        pl.kernel,
        out_type=(jax.ShapeDtypeStruct((NC, 2, N_PAD), jnp.float32),
                  jax.ShapeDtypeStruct((NC, 2, N_PAD), jnp.float32)),
        mesh=plsc.VectorSubcoreMesh(core_axis_name="c", subcore_axis_name="s",
                                    num_cores=NC, num_subcores=NS),
        scratch_types=[
            pltpu.VMEM_SHARED((N_PAD,), jnp.float32),
            pltpu.VMEM_SHARED((N_PAD,), jnp.float32),
            pltpu.VMEM_SHARED((N_PAD,), jnp.float32),
            pltpu.VMEM_SHARED((N_PAD,), jnp.float32),
            pltpu.VMEM((R, GROUP), jnp.int32),
            pltpu.VMEM((R, GROUP), jnp.int32),
            pltpu.VMEM((R, GROUP), jnp.float32),
            pltpu.VMEM((R, GROUP), jnp.float32),
            pltpu.VMEM((N_PAD // NS,), jnp.float32),
            pltpu.SemaphoreType.DMA,
            pltpu.SemaphoreType.DMA,
        ],
        compiler_params=pltpu.CompilerParams(use_tc_tiling_on_sc=False),
    )(_sc_body)
    _SC_KERNEL_CACHE.append(k)
    return k


def _pad2(x):
    n = x.shape[0]
    return jnp.pad(x, (0, N_PAD - n)).reshape(ROWS, 128)


def kernel(lbls, no_lbl_idx, knn_sc, knn_fc, train_idx):
    n = lbls.shape[0]
    e = knn_sc.shape[1]
    g = e // GROUP

    lbls2 = _pad2(lbls.astype(jnp.float32))
    train2 = _pad2(train_idx.astype(jnp.float32))
    null2 = _pad2(no_lbl_idx.astype(jnp.float32))

    val2, af2 = pl.pallas_call(
        _prep_body,
        out_shape=(jax.ShapeDtypeStruct((ROWS, 128), jnp.float32),
                   jax.ShapeDtypeStruct((ROWS, 128), jnp.float32)),
    )(lbls2, train2, null2)
    val_t = val2.reshape(-1)
    af_t = af2.reshape(-1)

    src1 = knn_sc[0].reshape(g, GROUP)
    dst1 = knn_sc[1].reshape(g, GROUP)
    src2 = knn_fc[0].reshape(g, GROUP)
    dst2 = knn_fc[1].reshape(g, GROUP)
    zeros = jnp.zeros((N_PAD // NS,), jnp.float32)

    p1, p2 = _get_sc_kernel()(val_t, af_t, src1, dst1, src2, dst2, zeros)

    def planes(p):
        return (p[0, 0].reshape(ROWS, 128), p[1, 0].reshape(ROWS, 128),
                p[0, 1].reshape(ROWS, 128), p[1, 1].reshape(ROWS, 128))

    s1a, s1b, c1a, c1b = planes(p1)
    s2a, s2b, c2a, c2b = planes(p2)

    out2d, mask2d = pl.pallas_call(
        _fin_body,
        out_shape=(jax.ShapeDtypeStruct((ROWS, 128), jnp.float32),
                   jax.ShapeDtypeStruct((ROWS, 128), jnp.int32)),
    )(lbls2, train2, null2, s1a, s1b, c1a, c1b, s2a, s2b, c2a, c2b)

    out = out2d.reshape(-1)[:n]
    mask = mask2d.reshape(-1)[:n].astype(bool)
    return (out, mask)


# speedup vs baseline: 455.4087x; 1.9028x over previous
import functools

import jax
import jax.numpy as jnp
from jax import lax
from jax.experimental import pallas as pl
from jax.experimental.pallas import tpu as pltpu
from jax.experimental.pallas import tpu_sc as plsc

N_PAD = 102400
ROWS = N_PAD // 128
CH = 2048
NC, NS = 2, 16
NW = NC * NS
CPT = 99
G_PAD = NW * CPT + 32


def _prep_body(lbls_ref, train_ref, null_ref, val_ref, af_ref):
    a = train_ref[...] * (1.0 - null_ref[...])
    val_ref[...] = lbls_ref[...] * a
    af_ref[...] = a


def _fin_body(lbls_ref, train_ref, null_ref,
              s1a_ref, s1b_ref, c1a_ref, c1b_ref,
              s2a_ref, s2b_ref, c2a_ref, c2b_ref,
              out_ref, mask_ref):
    lbls = lbls_ref[...]
    train = train_ref[...] > 0.0
    b = train & (null_ref[...] > 0.0)

    def mean_or_lbls(s, c):
        has = b & (c > 0.0)
        return jnp.where(has, s / jnp.maximum(c, 1.0), lbls)

    l1 = mean_or_lbls(s1a_ref[...] + s1b_ref[...], c1a_ref[...] + c1b_ref[...])
    l2 = mean_or_lbls(s2a_ref[...] + s2b_ref[...], c2a_ref[...] + c2b_ref[...])
    fin = (jnp.abs(l1) != jnp.inf) & (jnp.abs(l2) != jnp.inf) & train
    out_ref[...] = jnp.where(fin, (l1 + l2) * 0.5, lbls)
    mask_ref[...] = fin.astype(jnp.int32)


def _sc_body(val_t, af_t, src1, dst1, src2, dst2, zeros,
             out1, out2,
             val_s, af_s,
             acc_s1, acc_c1, acc_s2, acc_c2,
             src_buf, dst_buf, vrows, arows, zbuf, sem_l, sem_g, sem_s):
    cid = lax.axis_index("c")
    sid = lax.axis_index("s")
    wid = sid * NC + cid

    per_tile_nodes = N_PAD // NS
    zslice = pl.ds(sid * per_tile_nodes, per_tile_nodes)

    pltpu.sync_copy(val_t.at[zslice], zbuf)
    pltpu.sync_copy(zbuf, val_s.at[zslice])
    pltpu.sync_copy(af_t.at[zslice], zbuf)
    pltpu.sync_copy(zbuf, af_s.at[zslice])
    pltpu.sync_copy(zeros, zbuf)
    for acc in (acc_s1, acc_c1, acc_s2, acc_c2):
        pltpu.sync_copy(zbuf, acc.at[zslice])
    plsc.subcore_barrier()

    def run_graph(srcg, dstg, acc_s, acc_c):
        base = wid * CPT

        def chunk_body(i, carry):
            c = base + i
            l1 = pltpu.async_copy(srcg.at[c], src_buf, sem_l)
            l2 = pltpu.async_copy(dstg.at[c], dst_buf, sem_l)
            l1.wait()
            l2.wait()
            g1 = pltpu.async_copy(val_s.at[src_buf], vrows, sem_g)
            g2 = pltpu.async_copy(af_s.at[src_buf], arows, sem_g)
            g1.wait()
            g2.wait()
            s1 = pltpu.async_copy(vrows, acc_s.at[dst_buf], sem_s, add=True)
            s2 = pltpu.async_copy(arows, acc_c.at[dst_buf], sem_s, add=True)
            s1.wait()
            s2.wait()
            return carry

        lax.fori_loop(0, CPT, chunk_body, 0)

    run_graph(src1, dst1, acc_s1, acc_c1)
    run_graph(src2, dst2, acc_s2, acc_c2)
    plsc.subcore_barrier()

    for k, acc in enumerate((acc_s1, acc_c1)):
        pltpu.sync_copy(acc.at[zslice], zbuf)
        pltpu.sync_copy(zbuf, out1.at[cid, k, zslice])
    for k, acc in enumerate((acc_s2, acc_c2)):
        pltpu.sync_copy(acc.at[zslice], zbuf)
        pltpu.sync_copy(zbuf, out2.at[cid, k, zslice])


_SC_KERNEL_CACHE = []


def _get_sc_kernel():
    if _SC_KERNEL_CACHE:
        return _SC_KERNEL_CACHE[0]
    k = functools.partial(
        pl.kernel,
        out_type=(jax.ShapeDtypeStruct((NC, 2, N_PAD), jnp.float32),
                  jax.ShapeDtypeStruct((NC, 2, N_PAD), jnp.float32)),
        mesh=plsc.VectorSubcoreMesh(core_axis_name="c", subcore_axis_name="s",
                                    num_cores=NC, num_subcores=NS),
        scratch_types=[
            pltpu.VMEM_SHARED((N_PAD,), jnp.float32),
            pltpu.VMEM_SHARED((N_PAD,), jnp.float32),
            pltpu.VMEM_SHARED((N_PAD,), jnp.float32),
            pltpu.VMEM_SHARED((N_PAD,), jnp.float32),
            pltpu.VMEM_SHARED((N_PAD,), jnp.float32),
            pltpu.VMEM_SHARED((N_PAD,), jnp.float32),
            pltpu.VMEM((CH,), jnp.int32),
            pltpu.VMEM((CH,), jnp.int32),
            pltpu.VMEM((CH,), jnp.float32),
            pltpu.VMEM((CH,), jnp.float32),
            pltpu.VMEM((N_PAD // NS,), jnp.float32),
            pltpu.SemaphoreType.DMA,
            pltpu.SemaphoreType.DMA,
            pltpu.SemaphoreType.DMA,
        ],
        compiler_params=pltpu.CompilerParams(use_tc_tiling_on_sc=False),
    )(_sc_body)
    _SC_KERNEL_CACHE.append(k)
    return k


def _pad2(x):
    n = x.shape[0]
    return jnp.pad(x, (0, N_PAD - n)).reshape(ROWS, 128)


def kernel(lbls, no_lbl_idx, knn_sc, knn_fc, train_idx):
    n = lbls.shape[0]
    e = knn_sc.shape[1]

    lbls2 = _pad2(lbls.astype(jnp.float32))
    train2 = _pad2(train_idx.astype(jnp.float32))
    null2 = _pad2(no_lbl_idx.astype(jnp.float32))

    val2, af2 = pl.pallas_call(
        _prep_body,
        out_shape=(jax.ShapeDtypeStruct((ROWS, 128), jnp.float32),
                   jax.ShapeDtypeStruct((ROWS, 128), jnp.float32)),
    )(lbls2, train2, null2)
    val_t = val2.reshape(-1)
    af_t = af2.reshape(-1)

    pad_e = G_PAD * CH - e

    def chunked(x):
        return jnp.concatenate(
            [x, jnp.full((pad_e,), n, jnp.int32)]).reshape(G_PAD, CH)

    src1 = chunked(knn_sc[0])
    dst1 = chunked(knn_sc[1])
    src2 = chunked(knn_fc[0])
    dst2 = chunked(knn_fc[1])
    zeros = jnp.zeros((N_PAD // NS,), jnp.float32)

    p1, p2 = _get_sc_kernel()(val_t, af_t, src1, dst1, src2, dst2, zeros)

    def planes(p):
        return (p[0, 0].reshape(ROWS, 128), p[1, 0].reshape(ROWS, 128),
                p[0, 1].reshape(ROWS, 128), p[1, 1].reshape(ROWS, 128))

    s1a, s1b, c1a, c1b = planes(p1)
    s2a, s2b, c2a, c2b = planes(p2)

    out2d, mask2d = pl.pallas_call(
        _fin_body,
        out_shape=(jax.ShapeDtypeStruct((ROWS, 128), jnp.float32),
                   jax.ShapeDtypeStruct((ROWS, 128), jnp.int32)),
    )(lbls2, train2, null2, s1a, s1b, c1a, c1b, s2a, s2b, c2a, c2b)

    out = out2d.reshape(-1)[:n]
    mask = mask2d.reshape(-1)[:n].astype(bool)
    return (out, mask)


# speedup vs baseline: 652.6378x; 1.4331x over previous
import functools

import jax
import jax.numpy as jnp
from jax import lax
from jax.experimental import pallas as pl
from jax.experimental.pallas import tpu as pltpu
from jax.experimental.pallas import tpu_sc as plsc

N_PAD = 102400
ROWS = N_PAD // 128
CH = 2048
NC, NS = 2, 16
NW = NC * NS
CPT = 99
G_PAD = NW * CPT + 32


def _prep_body(lbls_ref, train_ref, null_ref, val_ref, af_ref):
    a = train_ref[...] * (1.0 - null_ref[...])
    val_ref[...] = lbls_ref[...] * a
    af_ref[...] = a


def _fin_body(lbls_ref, train_ref, null_ref,
              s1a_ref, s1b_ref, c1a_ref, c1b_ref,
              s2a_ref, s2b_ref, c2a_ref, c2b_ref,
              out_ref, mask_ref):
    lbls = lbls_ref[...]
    train = train_ref[...] > 0.0
    b = train & (null_ref[...] > 0.0)

    def mean_or_lbls(s, c):
        has = b & (c > 0.0)
        return jnp.where(has, s / jnp.maximum(c, 1.0), lbls)

    l1 = mean_or_lbls(s1a_ref[...] + s1b_ref[...], c1a_ref[...] + c1b_ref[...])
    l2 = mean_or_lbls(s2a_ref[...] + s2b_ref[...], c2a_ref[...] + c2b_ref[...])
    fin = (jnp.abs(l1) != jnp.inf) & (jnp.abs(l2) != jnp.inf) & train
    out_ref[...] = jnp.where(fin, (l1 + l2) * 0.5, lbls)
    mask_ref[...] = fin.astype(jnp.int32)


def _sc_body(val_t, af_t, src1, dst1, src2, dst2, zeros,
             out1, out2,
             val_s, af_s,
             acc_s1, acc_c1, acc_s2, acc_c2,
             src_buf, dst_buf, vrows, arows, zbuf,
             sem_l, sem_g, sem_s0, sem_s1, sem_s2):
    cid = lax.axis_index("c")
    sid = lax.axis_index("s")
    wid = sid * NC + cid

    per_tile_nodes = N_PAD // NS
    zslice = pl.ds(sid * per_tile_nodes, per_tile_nodes)

    pltpu.sync_copy(val_t.at[zslice], zbuf)
    pltpu.sync_copy(zbuf, val_s.at[zslice])
    pltpu.sync_copy(af_t.at[zslice], zbuf)
    pltpu.sync_copy(zbuf, af_s.at[zslice])
    pltpu.sync_copy(zeros, zbuf)
    for acc in (acc_s1, acc_c1, acc_s2, acc_c2):
        pltpu.sync_copy(zbuf, acc.at[zslice])
    plsc.subcore_barrier()

    ssems = (sem_s0, sem_s1, sem_s2)

    def run_graph(srcg, dstg, acc_s, acc_c):
        base = wid * CPT

        def fire_loads(c, s):
            pltpu.async_copy(srcg.at[base + c], src_buf.at[s], sem_l)
            pltpu.async_copy(dstg.at[base + c], dst_buf.at[s], sem_l)

        def drain_loads(c, s):
            pltpu.make_async_copy(srcg.at[base + c], src_buf.at[s], sem_l).wait()
            pltpu.make_async_copy(dstg.at[base + c], dst_buf.at[s], sem_l).wait()

        def fire_scatters(s):
            pltpu.async_copy(vrows.at[s], acc_s.at[dst_buf.at[s]], ssems[s],
                             add=True)
            pltpu.async_copy(arows.at[s], acc_c.at[dst_buf.at[s]], ssems[s],
                             add=True)

        def drain_scatters(s):
            pltpu.make_async_copy(vrows.at[s], acc_s.at[dst_buf.at[s]],
                                  ssems[s]).wait()
            pltpu.make_async_copy(arows.at[s], acc_c.at[dst_buf.at[s]],
                                  ssems[s]).wait()

        def step(i, k, first_block):
            s = k
            nxt = (k + 1) % 3
            drain_loads(i, s)
            if not (first_block and k < 2):
                drain_scatters(nxt)
            fire_loads(i + 1, nxt)
            g1 = pltpu.async_copy(val_s.at[src_buf.at[s]], vrows.at[s], sem_g)
            g2 = pltpu.async_copy(af_s.at[src_buf.at[s]], arows.at[s], sem_g)
            g1.wait()
            g2.wait()
            fire_scatters(s)

        fire_loads(0, 0)
        for k in range(3):
            step(k, k, True)

        def blk(i3, carry):
            for k in range(3):
                step(3 * i3 + k, k, False)
            return carry

        lax.fori_loop(1, CPT // 3, blk, 0)
        drain_scatters(1)
        drain_scatters(2)
        drain_loads(CPT, 0)

    run_graph(src1, dst1, acc_s1, acc_c1)
    run_graph(src2, dst2, acc_s2, acc_c2)
    plsc.subcore_barrier()

    for k, acc in enumerate((acc_s1, acc_c1)):
        pltpu.sync_copy(acc.at[zslice], zbuf)
        pltpu.sync_copy(zbuf, out1.at[cid, k, zslice])
    for k, acc in enumerate((acc_s2, acc_c2)):
        pltpu.sync_copy(acc.at[zslice], zbuf)
        pltpu.sync_copy(zbuf, out2.at[cid, k, zslice])


_SC_KERNEL_CACHE = []


def _get_sc_kernel():
    if _SC_KERNEL_CACHE:
        return _SC_KERNEL_CACHE[0]
    k = functools.partial(
        pl.kernel,
        out_type=(jax.ShapeDtypeStruct((NC, 2, N_PAD), jnp.float32),
                  jax.ShapeDtypeStruct((NC, 2, N_PAD), jnp.float32)),
        mesh=plsc.VectorSubcoreMesh(core_axis_name="c", subcore_axis_name="s",
                                    num_cores=NC, num_subcores=NS),
        scratch_types=[
            pltpu.VMEM_SHARED((N_PAD,), jnp.float32),
            pltpu.VMEM_SHARED((N_PAD,), jnp.float32),
            pltpu.VMEM_SHARED((N_PAD,), jnp.float32),
            pltpu.VMEM_SHARED((N_PAD,), jnp.float32),
            pltpu.VMEM_SHARED((N_PAD,), jnp.float32),
            pltpu.VMEM_SHARED((N_PAD,), jnp.float32),
            pltpu.VMEM((3, CH), jnp.int32),
            pltpu.VMEM((3, CH), jnp.int32),
            pltpu.VMEM((3, CH), jnp.float32),
            pltpu.VMEM((3, CH), jnp.float32),
            pltpu.VMEM((N_PAD // NS,), jnp.float32),
            pltpu.SemaphoreType.DMA,
            pltpu.SemaphoreType.DMA,
            pltpu.SemaphoreType.DMA,
            pltpu.SemaphoreType.DMA,
            pltpu.SemaphoreType.DMA,
        ],
        compiler_params=pltpu.CompilerParams(use_tc_tiling_on_sc=False),
    )(_sc_body)
    _SC_KERNEL_CACHE.append(k)
    return k


def _pad2(x):
    n = x.shape[0]
    return jnp.pad(x, (0, N_PAD - n)).reshape(ROWS, 128)


def kernel(lbls, no_lbl_idx, knn_sc, knn_fc, train_idx):
    n = lbls.shape[0]
    e = knn_sc.shape[1]

    lbls2 = _pad2(lbls.astype(jnp.float32))
    train2 = _pad2(train_idx.astype(jnp.float32))
    null2 = _pad2(no_lbl_idx.astype(jnp.float32))

    val2, af2 = pl.pallas_call(
        _prep_body,
        out_shape=(jax.ShapeDtypeStruct((ROWS, 128), jnp.float32),
                   jax.ShapeDtypeStruct((ROWS, 128), jnp.float32)),
    )(lbls2, train2, null2)
    val_t = val2.reshape(-1)
    af_t = af2.reshape(-1)

    pad_e = G_PAD * CH - e

    def chunked(x):
        return jnp.concatenate(
            [x, jnp.full((pad_e,), n, jnp.int32)]).reshape(G_PAD, CH)

    src1 = chunked(knn_sc[0])
    dst1 = chunked(knn_sc[1])
    src2 = chunked(knn_fc[0])
    dst2 = chunked(knn_fc[1])
    zeros = jnp.zeros((N_PAD // NS,), jnp.float32)

    p1, p2 = _get_sc_kernel()(val_t, af_t, src1, dst1, src2, dst2, zeros)

    def planes(p):
        return (p[0, 0].reshape(ROWS, 128), p[1, 0].reshape(ROWS, 128),
                p[0, 1].reshape(ROWS, 128), p[1, 1].reshape(ROWS, 128))

    s1a, s1b, c1a, c1b = planes(p1)
    s2a, s2b, c2a, c2b = planes(p2)

    out2d, mask2d = pl.pallas_call(
        _fin_body,
        out_shape=(jax.ShapeDtypeStruct((ROWS, 128), jnp.float32),
                   jax.ShapeDtypeStruct((ROWS, 128), jnp.int32)),
    )(lbls2, train2, null2, s1a, s1b, c1a, c1b, s2a, s2b, c2a, c2b)

    out = out2d.reshape(-1)[:n]
    mask = mask2d.reshape(-1)[:n].astype(bool)
    return (out, mask)


# speedup vs baseline: 950.7970x; 1.4569x over previous
import functools

import jax
import jax.numpy as jnp
from jax import lax
from jax.experimental import pallas as pl
from jax.experimental.pallas import tpu as pltpu
from jax.experimental.pallas import tpu_sc as plsc

N_PAD = 102400
ROWS = N_PAD // 128
CH = 2000
NC, NS = 2, 16
NW = NC * NS
CPT = 100


def _prep_body(lbls_ref, train_ref, null_ref, val_ref, af_ref):
    a = train_ref[...] * (1.0 - null_ref[...])
    val_ref[...] = lbls_ref[...] * a
    af_ref[...] = a


def _fin_body(lbls_ref, train_ref, null_ref,
              s1a_ref, s1b_ref, c1a_ref, c1b_ref,
              s2a_ref, s2b_ref, c2a_ref, c2b_ref,
              out_ref, mask_ref):
    lbls = lbls_ref[...]
    train = train_ref[...] > 0.0
    b = train & (null_ref[...] > 0.0)

    def mean_or_lbls(s, c):
        has = b & (c > 0.0)
        return jnp.where(has, s / jnp.maximum(c, 1.0), lbls)

    l1 = mean_or_lbls(s1a_ref[...] + s1b_ref[...], c1a_ref[...] + c1b_ref[...])
    l2 = mean_or_lbls(s2a_ref[...] + s2b_ref[...], c2a_ref[...] + c2b_ref[...])
    fin = (jnp.abs(l1) != jnp.inf) & (jnp.abs(l2) != jnp.inf) & train
    out_ref[...] = jnp.where(fin, (l1 + l2) * 0.5, lbls)
    mask_ref[...] = fin.astype(jnp.int32)


def _sc_body(val_t, af_t, src1, dst1, src2, dst2, zeros,
             out1, out2,
             val_s, af_s,
             acc_s1, acc_c1, acc_s2, acc_c2,
             src_buf, dst_buf, vrows, arows, zbuf,
             sem_l, sem_g, sem_s0, sem_s1, sem_s2):
    cid = lax.axis_index("c")
    sid = lax.axis_index("s")
    wid = sid * NC + cid

    per_tile_nodes = N_PAD // NS
    zslice = pl.ds(sid * per_tile_nodes, per_tile_nodes)

    pltpu.sync_copy(val_t.at[zslice], zbuf)
    pltpu.sync_copy(zbuf, val_s.at[zslice])
    pltpu.sync_copy(af_t.at[zslice], zbuf)
    pltpu.sync_copy(zbuf, af_s.at[zslice])
    pltpu.sync_copy(zeros, zbuf)
    for acc in (acc_s1, acc_c1, acc_s2, acc_c2):
        pltpu.sync_copy(zbuf, acc.at[zslice])
    plsc.subcore_barrier()

    ssems = (sem_s0, sem_s1, sem_s2)

    def run_graph(srcg, dstg, acc_s, acc_c):
        base = wid * CPT
        last = srcg.shape[0] - 1

        def fire_loads(c, s):
            cc = jnp.minimum(base + c, last)
            pltpu.async_copy(srcg.at[cc], src_buf.at[s], sem_l)
            pltpu.async_copy(dstg.at[cc], dst_buf.at[s], sem_l)

        def drain_loads(c, s):
            cc = jnp.minimum(base + c, last)
            pltpu.make_async_copy(srcg.at[cc], src_buf.at[s], sem_l).wait()
            pltpu.make_async_copy(dstg.at[cc], dst_buf.at[s], sem_l).wait()

        def fire_scatters(s):
            pltpu.async_copy(vrows.at[s], acc_s.at[dst_buf.at[s]], ssems[s],
                             add=True)
            pltpu.async_copy(arows.at[s], acc_c.at[dst_buf.at[s]], ssems[s],
                             add=True)

        def drain_scatters(s):
            pltpu.make_async_copy(vrows.at[s], acc_s.at[dst_buf.at[s]],
                                  ssems[s]).wait()
            pltpu.make_async_copy(arows.at[s], acc_c.at[dst_buf.at[s]],
                                  ssems[s]).wait()

        def step(i, k, first_block):
            s = k
            nxt = (k + 1) % 3
            drain_loads(i, s)
            if not (first_block and k < 2):
                drain_scatters(nxt)
            fire_loads(i + 1, nxt)
            g1 = pltpu.async_copy(val_s.at[src_buf.at[s]], vrows.at[s], sem_g)
            g2 = pltpu.async_copy(af_s.at[src_buf.at[s]], arows.at[s], sem_g)
            g1.wait()
            g2.wait()
            fire_scatters(s)

        fire_loads(0, 0)
        for k in range(3):
            step(k, k, True)

        def blk(i3, carry):
            for k in range(3):
                step(3 * i3 + k, k, False)
            return carry

        lax.fori_loop(1, 33, blk, 0)
        step(99, 0, False)
        drain_scatters(2)
        drain_scatters(0)
        drain_loads(CPT, 1)

    run_graph(src1, dst1, acc_s1, acc_c1)
    run_graph(src2, dst2, acc_s2, acc_c2)
    plsc.subcore_barrier()

    for k, acc in enumerate((acc_s1, acc_c1)):
        pltpu.sync_copy(acc.at[zslice], zbuf)
        pltpu.sync_copy(zbuf, out1.at[cid, k, zslice])
    for k, acc in enumerate((acc_s2, acc_c2)):
        pltpu.sync_copy(acc.at[zslice], zbuf)
        pltpu.sync_copy(zbuf, out2.at[cid, k, zslice])


_SC_KERNEL_CACHE = []


def _get_sc_kernel():
    if _SC_KERNEL_CACHE:
        return _SC_KERNEL_CACHE[0]
    k = functools.partial(
        pl.kernel,
        out_type=(jax.ShapeDtypeStruct((NC, 2, N_PAD), jnp.float32),
                  jax.ShapeDtypeStruct((NC, 2, N_PAD), jnp.float32)),
        mesh=plsc.VectorSubcoreMesh(core_axis_name="c", subcore_axis_name="s",
                                    num_cores=NC, num_subcores=NS),
        scratch_types=[
            pltpu.VMEM_SHARED((N_PAD,), jnp.float32),
            pltpu.VMEM_SHARED((N_PAD,), jnp.float32),
            pltpu.VMEM_SHARED((N_PAD,), jnp.float32),
            pltpu.VMEM_SHARED((N_PAD,), jnp.float32),
            pltpu.VMEM_SHARED((N_PAD,), jnp.float32),
            pltpu.VMEM_SHARED((N_PAD,), jnp.float32),
            pltpu.VMEM((3, CH), jnp.int32),
            pltpu.VMEM((3, CH), jnp.int32),
            pltpu.VMEM((3, CH), jnp.float32),
            pltpu.VMEM((3, CH), jnp.float32),
            pltpu.VMEM((N_PAD // NS,), jnp.float32),
            pltpu.SemaphoreType.DMA,
            pltpu.SemaphoreType.DMA,
            pltpu.SemaphoreType.DMA,
            pltpu.SemaphoreType.DMA,
            pltpu.SemaphoreType.DMA,
        ],
        compiler_params=pltpu.CompilerParams(use_tc_tiling_on_sc=False),
    )(_sc_body)
    _SC_KERNEL_CACHE.append(k)
    return k


def _pad2(x):
    n = x.shape[0]
    return jnp.pad(x, (0, N_PAD - n)).reshape(ROWS, 128)


def kernel(lbls, no_lbl_idx, knn_sc, knn_fc, train_idx):
    n = lbls.shape[0]
    e = knn_sc.shape[1]

    lbls2 = _pad2(lbls.astype(jnp.float32))
    train2 = _pad2(train_idx.astype(jnp.float32))
    null2 = _pad2(no_lbl_idx.astype(jnp.float32))

    val2, af2 = pl.pallas_call(
        _prep_body,
        out_shape=(jax.ShapeDtypeStruct((ROWS, 128), jnp.float32),
                   jax.ShapeDtypeStruct((ROWS, 128), jnp.float32)),
    )(lbls2, train2, null2)
    val_t = val2.reshape(-1)
    af_t = af2.reshape(-1)

    def chunked(x):
        return x.reshape(e // CH, CH)

    src1 = chunked(knn_sc[0])
    dst1 = chunked(knn_sc[1])
    src2 = chunked(knn_fc[0])
    dst2 = chunked(knn_fc[1])
    zeros = jnp.zeros((N_PAD // NS,), jnp.float32)

    p1, p2 = _get_sc_kernel()(val_t, af_t, src1, dst1, src2, dst2, zeros)

    def planes(p):
        return (p[0, 0].reshape(ROWS, 128), p[1, 0].reshape(ROWS, 128),
                p[0, 1].reshape(ROWS, 128), p[1, 1].reshape(ROWS, 128))

    s1a, s1b, c1a, c1b = planes(p1)
    s2a, s2b, c2a, c2b = planes(p2)

    out2d, mask2d = pl.pallas_call(
        _fin_body,
        out_shape=(jax.ShapeDtypeStruct((ROWS, 128), jnp.float32),
                   jax.ShapeDtypeStruct((ROWS, 128), jnp.int32)),
    )(lbls2, train2, null2, s1a, s1b, c1a, c1b, s2a, s2b, c2a, c2b)

    out = out2d.reshape(-1)[:n]
    mask = mask2d.reshape(-1)[:n].astype(bool)
    return (out, mask)


# speedup vs baseline: 983.4303x; 1.0343x over previous
import functools

import jax
import jax.numpy as jnp
from jax import lax
from jax.experimental import pallas as pl
from jax.experimental.pallas import tpu as pltpu
from jax.experimental.pallas import tpu_sc as plsc

N_PAD = 102400
ROWS = N_PAD // 128
CH = 4000
NC, NS = 2, 16
NW = NC * NS
CPT = 50


def _prep_body(lbls_ref, train_ref, null_ref, val_ref, af_ref):
    a = train_ref[...] * (1.0 - null_ref[...])
    val_ref[...] = lbls_ref[...] * a
    af_ref[...] = a


def _fin_body(lbls_ref, train_ref, null_ref,
              s1a_ref, s1b_ref, c1a_ref, c1b_ref,
              s2a_ref, s2b_ref, c2a_ref, c2b_ref,
              out_ref, mask_ref):
    lbls = lbls_ref[...]
    train = train_ref[...] > 0.0
    b = train & (null_ref[...] > 0.0)

    def mean_or_lbls(s, c):
        has = b & (c > 0.0)
        return jnp.where(has, s / jnp.maximum(c, 1.0), lbls)

    l1 = mean_or_lbls(s1a_ref[...] + s1b_ref[...], c1a_ref[...] + c1b_ref[...])
    l2 = mean_or_lbls(s2a_ref[...] + s2b_ref[...], c2a_ref[...] + c2b_ref[...])
    fin = (jnp.abs(l1) != jnp.inf) & (jnp.abs(l2) != jnp.inf) & train
    out_ref[...] = jnp.where(fin, (l1 + l2) * 0.5, lbls)
    mask_ref[...] = fin.astype(jnp.int32)


def _sc_body(val_t, af_t, src1, dst1, src2, dst2, zeros,
             out1, out2,
             val_s, af_s,
             acc_s1, acc_c1, acc_s2, acc_c2,
             src_buf, dst_buf, vrows, arows, zbuf,
             sem_l, sem_g, sem_s0, sem_s1, sem_s2):
    cid = lax.axis_index("c")
    sid = lax.axis_index("s")
    wid = sid * NC + cid

    per_tile_nodes = N_PAD // NS
    zslice = pl.ds(sid * per_tile_nodes, per_tile_nodes)

    pltpu.sync_copy(val_t.at[zslice], zbuf)
    pltpu.sync_copy(zbuf, val_s.at[zslice])
    pltpu.sync_copy(af_t.at[zslice], zbuf)
    pltpu.sync_copy(zbuf, af_s.at[zslice])
    pltpu.sync_copy(zeros, zbuf)
    for acc in (acc_s1, acc_c1, acc_s2, acc_c2):
        pltpu.sync_copy(zbuf, acc.at[zslice])
    plsc.subcore_barrier()

    ssems = (sem_s0, sem_s1, sem_s2)

    def run_graph(srcg, dstg, acc_s, acc_c):
        base = wid * CPT
        last = srcg.shape[0] - 1

        def fire_loads(c, s):
            cc = jnp.minimum(base + c, last)
            pltpu.async_copy(srcg.at[cc], src_buf.at[s], sem_l)
            pltpu.async_copy(dstg.at[cc], dst_buf.at[s], sem_l)

        def drain_loads(c, s):
            cc = jnp.minimum(base + c, last)
            pltpu.make_async_copy(srcg.at[cc], src_buf.at[s], sem_l).wait()
            pltpu.make_async_copy(dstg.at[cc], dst_buf.at[s], sem_l).wait()

        def fire_scatters(s):
            pltpu.async_copy(vrows.at[s], acc_s.at[dst_buf.at[s]], ssems[s],
                             add=True)
            pltpu.async_copy(arows.at[s], acc_c.at[dst_buf.at[s]], ssems[s],
                             add=True)

        def drain_scatters(s):
            pltpu.make_async_copy(vrows.at[s], acc_s.at[dst_buf.at[s]],
                                  ssems[s]).wait()
            pltpu.make_async_copy(arows.at[s], acc_c.at[dst_buf.at[s]],
                                  ssems[s]).wait()

        def step(i, k, first_block):
            s = k
            nxt = (k + 1) % 3
            drain_loads(i, s)
            if not (first_block and k < 2):
                drain_scatters(nxt)
            fire_loads(i + 1, nxt)
            g1 = pltpu.async_copy(val_s.at[src_buf.at[s]], vrows.at[s], sem_g)
            g2 = pltpu.async_copy(af_s.at[src_buf.at[s]], arows.at[s], sem_g)
            g1.wait()
            g2.wait()
            fire_scatters(s)

        fire_loads(0, 0)
        for k in range(3):
            step(k, k, True)

        def blk(i3, carry):
            for k in range(3):
                step(3 * i3 + k, k, False)
            return carry

        lax.fori_loop(1, 16, blk, 0)
        step(48, 0, False)
        step(49, 1, False)
        drain_scatters(0)
        drain_scatters(1)
        drain_loads(CPT, 2)

    run_graph(src1, dst1, acc_s1, acc_c1)
    run_graph(src2, dst2, acc_s2, acc_c2)
    plsc.subcore_barrier()

    for k, acc in enumerate((acc_s1, acc_c1)):
        pltpu.sync_copy(acc.at[zslice], zbuf)
        pltpu.sync_copy(zbuf, out1.at[cid, k, zslice])
    for k, acc in enumerate((acc_s2, acc_c2)):
        pltpu.sync_copy(acc.at[zslice], zbuf)
        pltpu.sync_copy(zbuf, out2.at[cid, k, zslice])


_SC_KERNEL_CACHE = []


def _get_sc_kernel():
    if _SC_KERNEL_CACHE:
        return _SC_KERNEL_CACHE[0]
    k = functools.partial(
        pl.kernel,
        out_type=(jax.ShapeDtypeStruct((NC, 2, N_PAD), jnp.float32),
                  jax.ShapeDtypeStruct((NC, 2, N_PAD), jnp.float32)),
        mesh=plsc.VectorSubcoreMesh(core_axis_name="c", subcore_axis_name="s",
                                    num_cores=NC, num_subcores=NS),
        scratch_types=[
            pltpu.VMEM_SHARED((N_PAD,), jnp.float32),
            pltpu.VMEM_SHARED((N_PAD,), jnp.float32),
            pltpu.VMEM_SHARED((N_PAD,), jnp.float32),
            pltpu.VMEM_SHARED((N_PAD,), jnp.float32),
            pltpu.VMEM_SHARED((N_PAD,), jnp.float32),
            pltpu.VMEM_SHARED((N_PAD,), jnp.float32),
            pltpu.VMEM((3, CH), jnp.int32),
            pltpu.VMEM((3, CH), jnp.int32),
            pltpu.VMEM((3, CH), jnp.float32),
            pltpu.VMEM((3, CH), jnp.float32),
            pltpu.VMEM((N_PAD // NS,), jnp.float32),
            pltpu.SemaphoreType.DMA,
            pltpu.SemaphoreType.DMA,
            pltpu.SemaphoreType.DMA,
            pltpu.SemaphoreType.DMA,
            pltpu.SemaphoreType.DMA,
        ],
        compiler_params=pltpu.CompilerParams(use_tc_tiling_on_sc=False),
    )(_sc_body)
    _SC_KERNEL_CACHE.append(k)
    return k


def _pad2(x):
    n = x.shape[0]
    return jnp.pad(x, (0, N_PAD - n)).reshape(ROWS, 128)


def kernel(lbls, no_lbl_idx, knn_sc, knn_fc, train_idx):
    n = lbls.shape[0]
    e = knn_sc.shape[1]

    lbls2 = _pad2(lbls.astype(jnp.float32))
    train2 = _pad2(train_idx.astype(jnp.float32))
    null2 = _pad2(no_lbl_idx.astype(jnp.float32))

    val2, af2 = pl.pallas_call(
        _prep_body,
        out_shape=(jax.ShapeDtypeStruct((ROWS, 128), jnp.float32),
                   jax.ShapeDtypeStruct((ROWS, 128), jnp.float32)),
    )(lbls2, train2, null2)
    val_t = val2.reshape(-1)
    af_t = af2.reshape(-1)

    def chunked(x):
        return x.reshape(e // CH, CH)

    src1 = chunked(knn_sc[0])
    dst1 = chunked(knn_sc[1])
    src2 = chunked(knn_fc[0])
    dst2 = chunked(knn_fc[1])
    zeros = jnp.zeros((N_PAD // NS,), jnp.float32)

    p1, p2 = _get_sc_kernel()(val_t, af_t, src1, dst1, src2, dst2, zeros)

    def planes(p):
        return (p[0, 0].reshape(ROWS, 128), p[1, 0].reshape(ROWS, 128),
                p[0, 1].reshape(ROWS, 128), p[1, 1].reshape(ROWS, 128))

    s1a, s1b, c1a, c1b = planes(p1)
    s2a, s2b, c2a, c2b = planes(p2)

    out2d, mask2d = pl.pallas_call(
        _fin_body,
        out_shape=(jax.ShapeDtypeStruct((ROWS, 128), jnp.float32),
                   jax.ShapeDtypeStruct((ROWS, 128), jnp.int32)),
    )(lbls2, train2, null2, s1a, s1b, c1a, c1b, s2a, s2b, c2a, c2b)

    out = out2d.reshape(-1)[:n]
    mask = mask2d.reshape(-1)[:n].astype(bool)
    return (out, mask)
